# Initial kernel scaffold; baseline (speedup 1.0000x reference)
import jax
import jax.numpy as jnp
from jax.experimental import pallas as pl


def kernel(x, edge_index_intra, edge_index_inter, pos, W_node_cov, b_node_cov, gamma_cov, beta_cov, W_node_ncov, b_node_ncov, gamma_ncov, beta_ncov, W_coord_cov, b_coord_cov, W_coord_ncov, b_coord_ncov):
    raise NotImplementedError("write your pallas kernel here")


# speedup vs baseline: 1.0136x; 1.0136x over previous
import jax
import jax.numpy as jnp
from jax.experimental import pallas as pl

_BN_SCALE = float((1.0 + 1e-5) ** -0.5)


def _mlp_pair_body(h_ref, oi_ref, on_ref, Wc_ref, bc_ref, gc_ref, btc_ref,
                   Wn_ref, bn_ref, gn_ref, btn_ref, out_ref):
    h = h_ref[...]
    zc = jnp.dot(h + oi_ref[...], Wc_ref[...],
                 preferred_element_type=jnp.float32) + bc_ref[...]
    zc = jnp.where(zc >= 0, zc, 0.2 * zc)
    zc = (zc * _BN_SCALE) * gc_ref[...] + btc_ref[...]
    zn = jnp.dot(h + on_ref[...], Wn_ref[...],
                 preferred_element_type=jnp.float32) + bn_ref[...]
    zn = jnp.where(zn >= 0, zn, 0.2 * zn)
    zn = (zn * _BN_SCALE) * gn_ref[...] + btn_ref[...]
    out_ref[...] = zc + zn


def _mlp_pair(h, oi, on, Wc, bc, gc, btc, Wn, bn, gn, btn):
    return pl.pallas_call(
        _mlp_pair_body,
        out_shape=jax.ShapeDtypeStruct(h.shape, h.dtype),
    )(h, oi, on, Wc, bc, gc, btc, Wn, bn, gn, btn)


def _rbf(dist):
    D_mu = jnp.linspace(0.0, 6.0, 9).reshape(1, -1)
    D_sigma = 6.0 / 9.0
    return jnp.exp(-((dist[:, None] - D_mu) / D_sigma) ** 2)


def _mean_aggr(msgs, dst, num_nodes):
    s = jax.ops.segment_sum(msgs, dst, num_segments=num_nodes)
    cnt = jax.ops.segment_sum(jnp.ones((msgs.shape[0],), msgs.dtype), dst,
                              num_segments=num_nodes)
    return s / jnp.clip(cnt, 1.0, None)[:, None]


def _branch(h, rbf_feats, ei, Wc, bc, num_nodes):
    row, col = ei[0], ei[1]
    radial = jax.nn.silu(rbf_feats @ Wc + bc)
    msg = h[row] * radial
    return _mean_aggr(msg, col, num_nodes)


def kernel(x, edge_index_intra, edge_index_inter, pos,
           W_node_cov, b_node_cov, gamma_cov, beta_cov,
           W_node_ncov, b_node_ncov, gamma_ncov, beta_ncov,
           W_coord_cov, b_coord_cov, W_coord_ncov, b_coord_ncov):
    n = x.shape[0]
    L = W_node_cov.shape[0]
    h = x

    def rbf_of(ei):
        cd = pos[ei[0]] - pos[ei[1]]
        dist = jnp.linalg.norm(cd, axis=-1)
        return _rbf(dist)

    rbf_intra = rbf_of(edge_index_intra)
    rbf_inter = rbf_of(edge_index_inter)

    for l in range(L):
        out_intra = _branch(h, rbf_intra, edge_index_intra,
                            W_coord_cov[l], b_coord_cov[l], n)
        out_inter = _branch(h, rbf_inter, edge_index_inter,
                            W_coord_ncov[l], b_coord_ncov[l], n)
        h = _mlp_pair(h, out_intra, out_inter,
                      W_node_cov[l], b_node_cov[l], gamma_cov[l], beta_cov[l],
                      W_node_ncov[l], b_node_ncov[l], gamma_ncov[l], beta_ncov[l])
    return h
